# baseline (speedup 1.0000x reference)
import functools

import jax
import jax.numpy as jnp
from jax import lax
from jax.experimental import pallas as pl
from jax.experimental.pallas import tpu as pltpu
from jax.experimental.pallas import tpu_sc as plsc

B, S, D, E, L, V = 128, 512, 768, 6, 2, 30000

_NW = 16
_RPW = B // _NW


@functools.cache
def _make_sc_gather():
    nc = 2
    mesh = plsc.VectorSubcoreMesh(
        core_axis_name="c", subcore_axis_name="s", num_cores=nc, num_subcores=16
    )

    @functools.partial(
        pl.kernel,
        mesh=mesh,
        out_type=jax.ShapeDtypeStruct((B, D), jnp.float32),
        scratch_types=[
            pltpu.VMEM((16,), jnp.int32),
            pltpu.VMEM((8, D), jnp.float32),
            pltpu.VMEM((_RPW, D), jnp.float32),
        ],
    )
    def sc_gather(idx_hbm, table_hbm, out_hbm, idx_v, blk_v, rows_v):
        wid = lax.axis_index("s") * nc + lax.axis_index("c")

        @pl.when(wid < _NW)
        def _():
            base = wid * _RPW
            pltpu.sync_copy(idx_hbm.at[pl.ds(base, _RPW)], idx_v.at[pl.ds(0, _RPW)])
            idxs = idx_v[...]
            for k in range(_RPW):
                r = idxs[k]
                blk = pl.multiple_of((r // 8) * 8, 8)
                pltpu.sync_copy(table_hbm.at[pl.ds(blk, 8)], blk_v)
                rm = r % 8

                def cp(j, carry, k=k, rm=rm):
                    rows_v[k, pl.ds(j * 16, 16)] = blk_v[rm, pl.ds(j * 16, 16)]
                    return carry

                lax.fori_loop(0, D // 16, cp, 0)
            pltpu.sync_copy(rows_v, out_hbm.at[pl.ds(base, _RPW)])

    return sc_gather


def _moe_head(cls_ref, mask_ref, gw_ref, gb_ref, ew_ref, eb_ref, out_ref):
    cls = cls_ref[...] * mask_ref[...]
    gl = jnp.dot(cls, gw_ref[...], preferred_element_type=jnp.float32) + gb_ref[...]
    eo = jnp.dot(cls, ew_ref[...], preferred_element_type=jnp.float32) + eb_ref[...]
    mx = jnp.max(gl, axis=1, keepdims=True)
    iota_e = lax.broadcasted_iota(jnp.int32, (B, E), 1)
    choice = jnp.min(jnp.where(gl >= mx, iota_e, E), axis=1, keepdims=True)
    iota_el = lax.broadcasted_iota(jnp.int32, (B, E * L), 1)
    o0 = jnp.sum(jnp.where(iota_el == L * choice, eo, 0.0), axis=1, keepdims=True)
    o1 = jnp.sum(jnp.where(iota_el == L * choice + 1, eo, 0.0), axis=1, keepdims=True)
    iota_l = lax.broadcasted_iota(jnp.int32, (B, L), 1)
    out_ref[...] = jnp.where(iota_l == 0, o0, o1)


def kernel(input_ids, attention_mask, embed_table, gate_W, gate_b, experts_W, experts_b):
    idx = input_ids[:, 0]
    mask_col = attention_mask[:, 0:1].astype(jnp.float32)
    ew2 = jnp.transpose(experts_W, (1, 0, 2)).reshape(D, E * L)
    gb2 = gate_b.reshape(1, E)
    eb2 = experts_b.reshape(1, E * L)

    cls_raw = _make_sc_gather()(idx, embed_table)

    return pl.pallas_call(
        _moe_head,
        out_shape=jax.ShapeDtypeStruct((B, L), jnp.float32),
    )(cls_raw, mask_col, gate_W, gb2, ew2, eb2)


# speedup vs baseline: 1.3486x; 1.3486x over previous
import functools

import jax
import jax.numpy as jnp
from jax import lax
from jax.experimental import pallas as pl
from jax.experimental.pallas import tpu as pltpu
from jax.experimental.pallas import tpu_sc as plsc

B, S, D, E, L, V = 128, 512, 768, 6, 2, 30000

_NW = 32
_RPW = B // _NW


@functools.cache
def _make_sc_gather():
    nc = 2
    mesh = plsc.VectorSubcoreMesh(
        core_axis_name="c", subcore_axis_name="s", num_cores=nc, num_subcores=16
    )

    @functools.partial(
        pl.kernel,
        mesh=mesh,
        out_type=jax.ShapeDtypeStruct((B, D), jnp.float32),
        scratch_types=[
            pltpu.VMEM((_RPW,), jnp.int32),
            pltpu.VMEM((_RPW, D), jnp.float32),
            pltpu.SemaphoreType.DMA,
        ],
    )
    def sc_gather(ids_hbm, table_hbm, out_hbm, idx_v, rows_v, sem):
        wid = lax.axis_index("s") * nc + lax.axis_index("c")
        base = wid * _RPW
        pltpu.sync_copy(ids_hbm.at[pl.ds(base, _RPW), 0], idx_v)
        pltpu.async_copy(table_hbm.at[idx_v], rows_v, sem).wait()
        pltpu.sync_copy(rows_v, out_hbm.at[pl.ds(base, _RPW)])

    return sc_gather


def _moe_head(cls_ref, mask_ref, gw_ref, gb_ref, ew_ref, eb_ref, out_ref):
    cls = cls_ref[...] * mask_ref[...]
    gl = jnp.dot(cls, gw_ref[...], preferred_element_type=jnp.float32) + gb_ref[...]
    eo = jnp.dot(cls, ew_ref[...], preferred_element_type=jnp.float32) + eb_ref[...]
    mx = jnp.max(gl, axis=1, keepdims=True)
    iota_e = lax.broadcasted_iota(jnp.int32, (B, E), 1)
    choice = jnp.min(jnp.where(gl >= mx, iota_e, E), axis=1, keepdims=True)
    iota_el = lax.broadcasted_iota(jnp.int32, (B, E * L), 1)
    o0 = jnp.sum(jnp.where(iota_el == L * choice, eo, 0.0), axis=1, keepdims=True)
    o1 = jnp.sum(jnp.where(iota_el == L * choice + 1, eo, 0.0), axis=1, keepdims=True)
    iota_l = lax.broadcasted_iota(jnp.int32, (B, L), 1)
    out_ref[...] = jnp.where(iota_l == 0, o0, o1)


def kernel(input_ids, attention_mask, embed_table, gate_W, gate_b, experts_W, experts_b):
    mask_col = attention_mask[:, 0:1].astype(jnp.float32)
    ew2 = jnp.transpose(experts_W, (1, 0, 2)).reshape(D, E * L)
    gb2 = gate_b.reshape(1, E)
    eb2 = experts_b.reshape(1, E * L)

    cls_raw = _make_sc_gather()(input_ids, embed_table)

    return pl.pallas_call(
        _moe_head,
        out_shape=jax.ShapeDtypeStruct((B, L), jnp.float32),
    )(cls_raw, mask_col, gate_W, gb2, ew2, eb2)


# speedup vs baseline: 1.3532x; 1.0034x over previous
import functools

import jax
import jax.numpy as jnp
from jax import lax
from jax.experimental import pallas as pl
from jax.experimental.pallas import tpu as pltpu
from jax.experimental.pallas import tpu_sc as plsc

B, S, D, E, L, V = 128, 512, 768, 6, 2, 30000

_NW = 32
_RPW = B // _NW


@functools.cache
def _make_sc_gather():
    nc = 2
    mesh = plsc.VectorSubcoreMesh(
        core_axis_name="c", subcore_axis_name="s", num_cores=nc, num_subcores=16
    )

    @functools.partial(
        pl.kernel,
        mesh=mesh,
        out_type=jax.ShapeDtypeStruct((B, D), jnp.float32),
        scratch_types=[
            pltpu.VMEM((_RPW,), jnp.int32),
            pltpu.VMEM((_RPW, D), jnp.float32),
            pltpu.SemaphoreType.DMA,
        ],
        compiler_params=pltpu.CompilerParams(skip_device_barrier=True),
    )
    def sc_gather(ids_hbm, table_hbm, out_hbm, idx_v, rows_v, sem):
        wid = lax.axis_index("s") * nc + lax.axis_index("c")
        base = wid * _RPW
        pltpu.sync_copy(ids_hbm.at[pl.ds(base, _RPW), 0], idx_v)
        pltpu.async_copy(table_hbm.at[idx_v], rows_v, sem).wait()
        pltpu.sync_copy(rows_v, out_hbm.at[pl.ds(base, _RPW)])

    return sc_gather


def _moe_head(cls_ref, mask_ref, gw_ref, gb_ref, ew_ref, eb_ref, out_ref):
    cls = cls_ref[...] * mask_ref[...]
    gl = jnp.dot(cls, gw_ref[...], preferred_element_type=jnp.float32) + gb_ref[...]
    eo = jnp.dot(cls, ew_ref[...], preferred_element_type=jnp.float32) + eb_ref[...]
    mx = jnp.max(gl, axis=1, keepdims=True)
    iota_e = lax.broadcasted_iota(jnp.int32, (B, E), 1)
    choice = jnp.min(jnp.where(gl >= mx, iota_e, E), axis=1, keepdims=True)
    iota_el = lax.broadcasted_iota(jnp.int32, (B, E * L), 1)
    o0 = jnp.sum(jnp.where(iota_el == L * choice, eo, 0.0), axis=1, keepdims=True)
    o1 = jnp.sum(jnp.where(iota_el == L * choice + 1, eo, 0.0), axis=1, keepdims=True)
    iota_l = lax.broadcasted_iota(jnp.int32, (B, L), 1)
    out_ref[...] = jnp.where(iota_l == 0, o0, o1)


def kernel(input_ids, attention_mask, embed_table, gate_W, gate_b, experts_W, experts_b):
    mask_col = attention_mask[:, 0:1].astype(jnp.float32)
    ew2 = jnp.transpose(experts_W, (1, 0, 2)).reshape(D, E * L)
    gb2 = gate_b.reshape(1, E)
    eb2 = experts_b.reshape(1, E * L)

    cls_raw = _make_sc_gather()(input_ids, embed_table)

    return pl.pallas_call(
        _moe_head,
        out_shape=jax.ShapeDtypeStruct((B, L), jnp.float32),
    )(cls_raw, mask_col, gate_W, gb2, ew2, eb2)
